# baseline (speedup 1.0000x reference)
import functools

import jax
import jax.numpy as jnp
from jax.experimental import pallas as pl
from jax.experimental.pallas import tpu as pltpu

_B, _C, _H, _W = 16, 256, 32, 32
_E = 8
_HW = _H * _W


def _moe_kernel(x_ref, k_ref, wg_ref, bg_ref, we_ref, out_ref):
    x_mat = x_ref[0].reshape(_C, _HW)

    pooled = jnp.mean(x_mat, axis=1, keepdims=True).T
    gate = jax.lax.dot_general(
        pooled, wg_ref[:], (((1,), (0,)), ((), ())),
        preferred_element_type=jnp.float32,
    ) + bg_ref[:]
    gate = gate - jnp.max(gate, axis=1, keepdims=True)
    eg = jnp.exp(gate)
    logits = eg / jnp.sum(eg, axis=1, keepdims=True)

    ids = jax.lax.broadcasted_iota(jnp.int32, (1, _E), 1)
    i1 = jnp.argmax(logits, axis=1, keepdims=True)
    masked = jnp.where(ids == i1, -jnp.inf, logits)
    i2 = jnp.argmax(masked, axis=1, keepdims=True)
    keep = (ids == i1) | (ids == i2)
    w = jnp.where(keep, logits, 0.0)

    w_comb = jnp.sum(w.reshape(_E, 1, 1) * we_ref[:], axis=0)

    y = jax.lax.dot_general(
        w_comb, x_mat, (((1,), (0,)), ((), ())),
        preferred_element_type=jnp.float32,
    )
    kvec = k_ref[0].reshape(_C, 1)
    out_ref[0] = (x_mat + y * kvec).reshape(_C, _H, _W)


@jax.jit
def kernel(x, K, Wg, bg, We):
    bg2 = bg.reshape(1, _E)
    grid_spec = pl.GridSpec(
        grid=(_B,),
        in_specs=[
            pl.BlockSpec((1, _C, _H, _W), lambda b: (b, 0, 0, 0)),
            pl.BlockSpec((1, _C, 1, 1), lambda b: (b, 0, 0, 0)),
            pl.BlockSpec((_C, _E), lambda b: (0, 0)),
            pl.BlockSpec((1, _E), lambda b: (0, 0)),
            pl.BlockSpec((_E, _C, _C), lambda b: (0, 0, 0)),
        ],
        out_specs=pl.BlockSpec((1, _C, _H, _W), lambda b: (b, 0, 0, 0)),
    )
    return pl.pallas_call(
        _moe_kernel,
        grid_spec=grid_spec,
        out_shape=jax.ShapeDtypeStruct((_B, _C, _H, _W), jnp.float32),
        compiler_params=pltpu.CompilerParams(
            dimension_semantics=("parallel",),
        ),
    )(x, K, Wg, bg2, We)


# speedup vs baseline: 2.7776x; 2.7776x over previous
import functools

import jax
import jax.numpy as jnp
from jax.experimental import pallas as pl
from jax.experimental.pallas import tpu as pltpu

_B, _C, _H, _W = 16, 256, 32, 32
_E = 8
_HW = _H * _W


def _moe_kernel(x_ref, k_ref, wg_ref, bg_ref, we_ref, out_ref):
    x_mat = x_ref[0]

    pooled = jnp.mean(x_mat, axis=1, keepdims=True).T
    gate = jax.lax.dot_general(
        pooled, wg_ref[:], (((1,), (0,)), ((), ())),
        preferred_element_type=jnp.float32,
    ) + bg_ref[:]
    gate = gate - jnp.max(gate, axis=1, keepdims=True)
    eg = jnp.exp(gate)
    logits = eg / jnp.sum(eg, axis=1, keepdims=True)

    ids = jax.lax.broadcasted_iota(jnp.int32, (1, _E), 1)
    i1 = jnp.argmax(logits, axis=1, keepdims=True)
    masked = jnp.where(ids == i1, -jnp.inf, logits)
    i2 = jnp.argmax(masked, axis=1, keepdims=True)
    keep = (ids == i1) | (ids == i2)
    w = jnp.where(keep, logits, 0.0)

    w_comb = jnp.sum(w.reshape(_E, 1, 1) * we_ref[:], axis=0)

    y = jax.lax.dot_general(
        w_comb, x_mat, (((1,), (0,)), ((), ())),
        preferred_element_type=jnp.float32,
    )
    kvec = k_ref[0]
    out_ref[0] = x_mat + y * kvec


@jax.jit
def kernel(x, K, Wg, bg, We):
    bg2 = bg.reshape(1, _E)
    x3 = x.reshape(_B, _C, _HW)
    k3 = K.reshape(_B, _C, 1)
    grid_spec = pl.GridSpec(
        grid=(_B,),
        in_specs=[
            pl.BlockSpec((1, _C, _HW), lambda b: (b, 0, 0)),
            pl.BlockSpec((1, _C, 1), lambda b: (b, 0, 0)),
            pl.BlockSpec((_C, _E), lambda b: (0, 0)),
            pl.BlockSpec((1, _E), lambda b: (0, 0)),
            pl.BlockSpec((_E, _C, _C), lambda b: (0, 0, 0)),
        ],
        out_specs=pl.BlockSpec((1, _C, _HW), lambda b: (b, 0, 0)),
    )
    out = pl.pallas_call(
        _moe_kernel,
        grid_spec=grid_spec,
        out_shape=jax.ShapeDtypeStruct((_B, _C, _HW), jnp.float32),
        compiler_params=pltpu.CompilerParams(
            dimension_semantics=("parallel",),
        ),
    )(x3, k3, Wg, bg2, We)
    return out.reshape(_B, _C, _H, _W)
